# baseline (speedup 1.0000x reference)
import functools

import jax
import jax.numpy as jnp
from jax import lax
from jax.experimental import pallas as pl
from jax.experimental.pallas import tpu as pltpu
from jax.experimental.pallas import tpu_sc as plsc

_RANK = 16
_LAMBD = 0.01
_BATCH = 16384
_NC, _NS, _L = 2, 16, 16
_NW = _NC * _NS
_BPW = _BATCH // _NW


def _sc_body(idx0_hbm, idx1_hbm, idx2_hbm, y_hbm, f0_hbm, f1_hbm, f2_hbm,
             out_hbm, idx0_v, idx1_v, idx2_v, y_v, r0_v, r1_v, r2_v, out_v,
             sem0, sem1, sem2):
    wid = lax.axis_index("s") * _NC + lax.axis_index("c")
    base = wid * _BPW
    pltpu.sync_copy(idx0_hbm.at[pl.ds(base, _BPW)], idx0_v)
    pltpu.sync_copy(idx1_hbm.at[pl.ds(base, _BPW)], idx1_v)
    pltpu.sync_copy(idx2_hbm.at[pl.ds(base, _BPW)], idx2_v)
    pltpu.sync_copy(y_hbm.at[pl.ds(base, _BPW)], y_v)
    c0 = pltpu.async_copy(f0_hbm.at[idx0_v], r0_v, sem0)
    c1 = pltpu.async_copy(f1_hbm.at[idx1_v], r1_v, sem1)
    c2 = pltpu.async_copy(f2_hbm.at[idx2_v], r2_v, sem2)
    c0.wait()
    c1.wait()
    c2.wait()

    lane = lax.iota(jnp.int32, _L)

    def body(g, carry):
        acc, reg = carry
        bvec = g * _L + lane
        yv = y_v[pl.ds(g * _L, _L)]
        inner = jnp.zeros((_L,), jnp.float32)
        for r in range(_RANK):
            rvec = jnp.full((_L,), r, jnp.int32)
            c0 = plsc.load_gather(r0_v, [bvec, rvec])
            c1 = plsc.load_gather(r1_v, [bvec, rvec])
            c2 = plsc.load_gather(r2_v, [bvec, rvec])
            inner = inner + c0 * c1 * c2
            reg = reg + c0 * c0 + c1 * c1 + c2 * c2
        e = inner - yv
        return acc + e * e, reg

    zero = jnp.zeros((_L,), jnp.float32)
    acc, reg = lax.fori_loop(0, _BPW // _L, body, (zero, zero))
    out_v[...] = _LAMBD * reg + acc
    pltpu.sync_copy(out_v, out_hbm.at[wid])


@jax.jit
def _partials(idx0, idx1, idx2, y, f0, f1, f2):
    mesh = plsc.VectorSubcoreMesh(core_axis_name="c", subcore_axis_name="s")
    return pl.kernel(
        _sc_body,
        out_type=jax.ShapeDtypeStruct((_NW, _L), jnp.float32),
        mesh=mesh,
        compiler_params=pltpu.CompilerParams(needs_layout_passes=False, use_tc_tiling_on_sc=False),
        scratch_types=[
            pltpu.VMEM((_BPW,), jnp.int32),
            pltpu.VMEM((_BPW,), jnp.int32),
            pltpu.VMEM((_BPW,), jnp.int32),
            pltpu.VMEM((_BPW,), jnp.float32),
            pltpu.VMEM((_BPW, _RANK), jnp.float32),
            pltpu.VMEM((_BPW, _RANK), jnp.float32),
            pltpu.VMEM((_BPW, _RANK), jnp.float32),
            pltpu.VMEM((_L,), jnp.float32),
            pltpu.SemaphoreType.DMA,
            pltpu.SemaphoreType.DMA,
            pltpu.SemaphoreType.DMA,
        ],
    )(idx0, idx1, idx2, y, f0, f1, f2)


def kernel(indices, y, factor0, factor1, factor2):
    idx0 = indices[:, 0]
    idx1 = indices[:, 1]
    idx2 = indices[:, 2]
    parts = _partials(idx0, idx1, idx2, y, factor0, factor1, factor2)
    return jnp.sum(parts)


# speedup vs baseline: 6.9432x; 6.9432x over previous
import functools

import jax
import jax.numpy as jnp
from jax import lax
from jax.experimental import pallas as pl
from jax.experimental.pallas import tpu as pltpu
from jax.experimental.pallas import tpu_sc as plsc

_RANK = 16
_LAMBD = 0.01
_BATCH = 16384
_NC, _NS, _L = 2, 16, 16
_NW = _NC * _NS
_BPW = _BATCH // _NW
_NB = 16
_NG = _BPW // _NB


def _sc_body(idx0_hbm, idx1_hbm, idx2_hbm, y_hbm, f0_hbm, f1_hbm, f2_hbm,
             out_hbm, idx0_v, idx1_v, idx2_v, y_v, slabs_v, out_v, sems):
    wid = lax.axis_index("s") * _NC + lax.axis_index("c")
    base = wid * _BPW
    pltpu.sync_copy(idx0_hbm.at[pl.ds(base, _BPW)], idx0_v)
    pltpu.sync_copy(idx1_hbm.at[pl.ds(base, _BPW)], idx1_v)
    pltpu.sync_copy(idx2_hbm.at[pl.ds(base, _BPW)], idx2_v)
    pltpu.sync_copy(y_hbm.at[pl.ds(base, _BPW)], y_v)

    tables = (f0_hbm, f1_hbm, f2_hbm)
    lane = lax.iota(jnp.int32, _L)

    def fire(j, ivs):
        for t in range(3):
            i = ivs[t][j]
            off = pl.multiple_of(
                lax.shift_left(lax.shift_right_logical(i, 7), 7), 128)
            pltpu.async_copy(
                tables[t].at[:, pl.ds(off, 128)],
                slabs_v.at[pl.ds((j * 3 + t) * _L, _L)],
                sems.at[j])

    def drain(j):
        for t in range(3):
            pltpu.make_async_copy(
                tables[t].at[:, pl.ds(0, 128)],
                slabs_v.at[pl.ds((j * 3 + t) * _L, _L)],
                sems.at[j]).wait()

    def load_block_indices(g):
        return (idx0_v[pl.ds(g * _NB, _NB)],
                idx1_v[pl.ds(g * _NB, _NB)],
                idx2_v[pl.ds(g * _NB, _NB)])

    ivs0 = load_block_indices(0)
    for j in range(_NB):
        fire(j, ivs0)

    def body(g, carry):
        acc, reg = carry
        ivs = load_block_indices(g)
        yv = y_v[pl.ds(g * _NB, _NB)]
        nxt = jnp.minimum(g + 1, _NG - 1)
        nivs = load_block_indices(nxt)
        svec = jnp.zeros((_L,), jnp.float32)
        for j in range(_NB):
            drain(j)
            v = []
            for t in range(3):
                col = jnp.broadcast_to(jnp.bitwise_and(ivs[t][j], 127), (_L,))
                row = (j * 3 + t) * _L + lane
                v.append(plsc.load_gather(slabs_v, [row, col]))
            p = v[0] * v[1] * v[2]
            svec = jnp.where(lane == j, jnp.sum(p), svec)
            reg = reg + v[0] * v[0] + v[1] * v[1] + v[2] * v[2]

            @pl.when(g < _NG - 1)
            def _():
                fire(j, nivs)

        e = svec - yv
        return acc + e * e, reg

    zero = jnp.zeros((_L,), jnp.float32)
    acc, reg = lax.fori_loop(0, _NG, body, (zero, zero))
    out_v[...] = _LAMBD * reg + acc
    pltpu.sync_copy(out_v, out_hbm.at[wid])


@jax.jit
def _partials(idx0, idx1, idx2, y, f0t, f1t, f2t):
    mesh = plsc.VectorSubcoreMesh(core_axis_name="c", subcore_axis_name="s")
    return pl.kernel(
        _sc_body,
        out_type=jax.ShapeDtypeStruct((_NW, _L), jnp.float32),
        mesh=mesh,
        compiler_params=pltpu.CompilerParams(needs_layout_passes=False),
        scratch_types=[
            pltpu.VMEM((_BPW,), jnp.int32),
            pltpu.VMEM((_BPW,), jnp.int32),
            pltpu.VMEM((_BPW,), jnp.int32),
            pltpu.VMEM((_BPW,), jnp.float32),
            pltpu.VMEM((_NB * 3 * _L, 128), jnp.float32),
            pltpu.VMEM((_L,), jnp.float32),
            pltpu.SemaphoreType.DMA((_NB,)),
        ],
    )(idx0, idx1, idx2, y, f0t, f1t, f2t)


def kernel(indices, y, factor0, factor1, factor2):
    idx0 = indices[:, 0]
    idx1 = indices[:, 1]
    idx2 = indices[:, 2]
    parts = _partials(idx0, idx1, idx2, y,
                      factor0.T, factor1.T, factor2.T)
    return jnp.sum(parts)


# speedup vs baseline: 6.9722x; 1.0042x over previous
import functools

import jax
import jax.numpy as jnp
from jax import lax
from jax.experimental import pallas as pl
from jax.experimental.pallas import tpu as pltpu
from jax.experimental.pallas import tpu_sc as plsc

_RANK = 16
_LAMBD = 0.01
_BATCH = 16384
_NC, _NS, _L = 2, 16, 16
_NW = _NC * _NS
_BPW = _BATCH // _NW
_NB = 16
_NG = _BPW // _NB


def _sc_body(idx0_hbm, idx1_hbm, idx2_hbm, y_hbm, f0_hbm, f1_hbm, f2_hbm,
             out_hbm, idx0_v, idx1_v, idx2_v, y_v, slabs_v, out_v, sems):
    wid = lax.axis_index("s") * _NC + lax.axis_index("c")
    base = wid * _BPW
    c0 = pltpu.async_copy(idx0_hbm.at[pl.ds(base, _BPW)], idx0_v, sems.at[0])
    c1 = pltpu.async_copy(idx1_hbm.at[pl.ds(base, _BPW)], idx1_v, sems.at[1])
    c2 = pltpu.async_copy(idx2_hbm.at[pl.ds(base, _BPW)], idx2_v, sems.at[2])
    c3 = pltpu.async_copy(y_hbm.at[pl.ds(base, _BPW)], y_v, sems.at[3])
    c0.wait()
    c1.wait()
    c2.wait()
    c3.wait()

    tables = (f0_hbm, f1_hbm, f2_hbm)
    lane = lax.iota(jnp.int32, _L)

    def fire(j, ivs):
        for t in range(3):
            i = ivs[t][j]
            off = pl.multiple_of(
                lax.shift_left(lax.shift_right_logical(i, 7), 7), 128)
            pltpu.async_copy(
                tables[t].at[:, pl.ds(off, 128)],
                slabs_v.at[pl.ds((j * 3 + t) * _L, _L)],
                sems.at[j])

    def drain(j):
        for t in range(3):
            pltpu.make_async_copy(
                tables[t].at[:, pl.ds(0, 128)],
                slabs_v.at[pl.ds((j * 3 + t) * _L, _L)],
                sems.at[j]).wait()

    def load_block_indices(g):
        return (idx0_v[pl.ds(g * _NB, _NB)],
                idx1_v[pl.ds(g * _NB, _NB)],
                idx2_v[pl.ds(g * _NB, _NB)])

    ivs0 = load_block_indices(0)
    for j in range(_NB):
        fire(j, ivs0)

    def body(g, carry):
        acc, reg = carry
        ivs = load_block_indices(g)
        yv = y_v[pl.ds(g * _NB, _NB)]
        nxt = jnp.minimum(g + 1, _NG - 1)
        nivs = load_block_indices(nxt)
        svec = jnp.zeros((_L,), jnp.float32)
        for j in range(_NB):
            drain(j)
            v = []
            for t in range(3):
                col = jnp.broadcast_to(jnp.bitwise_and(ivs[t][j], 127), (_L,))
                row = (j * 3 + t) * _L + lane
                v.append(plsc.load_gather(slabs_v, [row, col]))
            p = v[0] * v[1] * v[2]
            svec = jnp.where(lane == j, jnp.sum(p), svec)
            reg = reg + v[0] * v[0] + v[1] * v[1] + v[2] * v[2]

            @pl.when(g < _NG - 1)
            def _():
                fire(j, nivs)

        e = svec - yv
        return acc + e * e, reg

    zero = jnp.zeros((_L,), jnp.float32)
    acc, reg = lax.fori_loop(0, _NG, body, (zero, zero))
    out_v[...] = _LAMBD * reg + acc
    pltpu.sync_copy(out_v, out_hbm.at[wid])


@jax.jit
def _partials(idx0, idx1, idx2, y, f0t, f1t, f2t):
    mesh = plsc.VectorSubcoreMesh(core_axis_name="c", subcore_axis_name="s")
    return pl.kernel(
        _sc_body,
        out_type=jax.ShapeDtypeStruct((_NW, _L), jnp.float32),
        mesh=mesh,
        compiler_params=pltpu.CompilerParams(needs_layout_passes=False),
        scratch_types=[
            pltpu.VMEM((_BPW,), jnp.int32),
            pltpu.VMEM((_BPW,), jnp.int32),
            pltpu.VMEM((_BPW,), jnp.int32),
            pltpu.VMEM((_BPW,), jnp.float32),
            pltpu.VMEM((_NB * 3 * _L, 128), jnp.float32),
            pltpu.VMEM((_L,), jnp.float32),
            pltpu.SemaphoreType.DMA((_NB,)),
        ],
    )(idx0, idx1, idx2, y, f0t, f1t, f2t)


def kernel(indices, y, factor0, factor1, factor2):
    idx0 = indices[:, 0]
    idx1 = indices[:, 1]
    idx2 = indices[:, 2]
    parts = _partials(idx0, idx1, idx2, y,
                      factor0.T, factor1.T, factor2.T)
    return jnp.sum(parts)


# speedup vs baseline: 7.0182x; 1.0066x over previous
import jax
import jax.numpy as jnp
from jax import lax
from jax.experimental import pallas as pl
from jax.experimental.pallas import tpu as pltpu
from jax.experimental.pallas import tpu_sc as plsc

_RANK = 16
_LAMBD = 0.01
_BATCH = 16384
_NC, _NS, _L = 2, 16, 16
_NW = _NC * _NS
_BPW = _BATCH // _NW
_NB = 16
_NG = _BPW // _NB


def _sc_body(idx0_hbm, idx1_hbm, idx2_hbm, y_hbm, f0_hbm, f1_hbm, f2_hbm,
             out_hbm, idx0_v, idx1_v, idx2_v, y_v, slabs_v, out_v, sems):
    wid = lax.axis_index("s") * _NC + lax.axis_index("c")
    base = wid * _BPW
    c0 = pltpu.async_copy(idx0_hbm.at[pl.ds(base, _BPW)], idx0_v, sems.at[0])
    c1 = pltpu.async_copy(idx1_hbm.at[pl.ds(base, _BPW)], idx1_v, sems.at[1])
    c2 = pltpu.async_copy(idx2_hbm.at[pl.ds(base, _BPW)], idx2_v, sems.at[2])
    c3 = pltpu.async_copy(y_hbm.at[pl.ds(base, _BPW)], y_v, sems.at[3])
    c0.wait()
    c1.wait()
    c2.wait()
    c3.wait()

    tables = (f0_hbm, f1_hbm, f2_hbm)
    lane = lax.iota(jnp.int32, _L)

    def fire(j, ivs):
        for t in range(3):
            i = ivs[t][j]
            off = pl.multiple_of(
                lax.shift_left(lax.shift_right_logical(i, 7), 7), 128)
            pltpu.async_copy(
                tables[t].at[:, pl.ds(off, 128)],
                slabs_v.at[pl.ds((j * 3 + t) * _L, _L)],
                sems.at[j])

    def drain(j):
        for t in range(3):
            pltpu.make_async_copy(
                tables[t].at[:, pl.ds(0, 128)],
                slabs_v.at[pl.ds((j * 3 + t) * _L, _L)],
                sems.at[j]).wait()

    def load_block_indices(g):
        return (idx0_v[pl.ds(g * _NB, _NB)],
                idx1_v[pl.ds(g * _NB, _NB)],
                idx2_v[pl.ds(g * _NB, _NB)])

    ivs0 = load_block_indices(0)
    for j in range(_NB):
        fire(j, ivs0)

    def body(g, carry):
        acc, reg = carry
        ivs = load_block_indices(g)
        yv = y_v[pl.ds(g * _NB, _NB)]
        nxt = jnp.minimum(g + 1, _NG - 1)
        nivs = load_block_indices(nxt)
        svec = jnp.zeros((_L,), jnp.float32)
        for j in range(_NB):
            drain(j)
            v = []
            for t in range(3):
                col = jnp.broadcast_to(jnp.bitwise_and(ivs[t][j], 127), (_L,))
                row = (j * 3 + t) * _L + lane
                v.append(plsc.load_gather(slabs_v, [row, col]))
            p = v[0] * v[1] * v[2]
            svec = jnp.where(lane == j, jnp.sum(p), svec)
            reg = reg + v[0] * v[0] + v[1] * v[1] + v[2] * v[2]

            @pl.when(g < _NG - 1)
            def _():
                fire(j, nivs)

        e = svec - yv
        return acc + e * e, reg

    zero = jnp.zeros((_L,), jnp.float32)
    acc, reg = lax.fori_loop(0, _NG, body, (zero, zero))
    out_v[...] = _LAMBD * reg + acc
    pltpu.sync_copy(out_v, out_hbm.at[wid])


@jax.jit
def _partials(idx0, idx1, idx2, y, f0t, f1t, f2t):
    mesh = plsc.VectorSubcoreMesh(core_axis_name="c", subcore_axis_name="s")
    return pl.kernel(
        _sc_body,
        out_type=jax.ShapeDtypeStruct((_NW, _L), jnp.float32),
        mesh=mesh,
        compiler_params=pltpu.CompilerParams(needs_layout_passes=False),
        scratch_types=[
            pltpu.VMEM((_BPW,), jnp.int32),
            pltpu.VMEM((_BPW,), jnp.int32),
            pltpu.VMEM((_BPW,), jnp.int32),
            pltpu.VMEM((_BPW,), jnp.float32),
            pltpu.VMEM((_NB * 3 * _L, 128), jnp.float32),
            pltpu.VMEM((_L,), jnp.float32),
            pltpu.SemaphoreType.DMA((_NB,)),
        ],
    )(idx0, idx1, idx2, y, f0t, f1t, f2t)


def kernel(indices, y, factor0, factor1, factor2):
    idx0 = indices[:, 0]
    idx1 = indices[:, 1]
    idx2 = indices[:, 2]
    parts = _partials(idx0, idx1, idx2, y,
                      factor0.T, factor1.T, factor2.T)
    return jnp.sum(parts)
